# Initial kernel scaffold; baseline (speedup 1.0000x reference)
import jax
import jax.numpy as jnp
from jax.experimental import pallas as pl


def kernel(x, edge_index, batch, params):
    raise NotImplementedError("write your pallas kernel here")


# speedup vs baseline: 1.7346x; 1.7346x over previous
import jax
import jax.numpy as jnp
from jax.experimental import pallas as pl

_B = 10
_NPER = 1000
_K0 = 100
_NEG = 0.2
_EPS = 1e-15


def _gat_sparse(x, src, dst, p, n):
    xl = x @ p["Wl"] + p["bl"]
    xr = x @ p["Wr"] + p["br"]
    e = xl[src] + xr[dst]
    e = jnp.where(e >= 0, e, _NEG * e)
    logits = e @ p["att"]
    m = jax.ops.segment_max(logits, dst, num_segments=n)
    m = jnp.where(jnp.isfinite(m), m, 0.0)
    ex = jnp.exp(logits - m[dst])
    denom = jax.ops.segment_sum(ex, dst, num_segments=n)
    alpha = ex / jnp.maximum(denom[dst], 1e-16)
    out = jax.ops.segment_sum(alpha[:, None] * xl[src], dst, num_segments=n)
    return out + p["bias"]


def _gat_dense(z, p):
    xl = z @ p["Wl"] + p["bl"]
    xr = z @ p["Wr"] + p["br"]
    e = xl[:, None, :, :] + xr[:, :, None, :]
    e = jnp.where(e >= 0, e, _NEG * e)
    logits = jnp.einsum("bijd,d->bij", e, p["att"])
    alpha = jax.nn.softmax(logits, axis=-1)
    return jnp.einsum("bij,bjd->bid", alpha, xl) + p["bias"]


def _heads_body(x_ref, w_ref, b_ref, o_ref):
    o_ref[...] = jnp.dot(x_ref[...], w_ref[...],
                         preferred_element_type=jnp.float32) + b_ref[...]


def _final_heads(x2, w_lin, b_lin, w_h1, b_h1):
    xp = jnp.zeros((16, 128), jnp.float32).at[:_B, :].set(x2)
    wp = jnp.zeros((128, 128), jnp.float32)
    wp = wp.at[:, 0:2].set(w_lin).at[:, 2:4].set(w_h1)
    bp = jnp.zeros((1, 128), jnp.float32)
    bp = bp.at[0, 0:2].set(b_lin).at[0, 2:4].set(b_h1)
    o = pl.pallas_call(
        _heads_body,
        out_shape=jax.ShapeDtypeStruct((16, 128), jnp.float32),
    )(xp, wp, bp)
    return o[:_B, 0:2], o[:_B, 2:4]


def kernel(x, edge_index, batch, params):
    del batch
    src, dst = edge_index[0], edge_index[1]
    n = x.shape[0]

    s = x
    for p in params["pool"][0]:
        s = _gat_sparse(s, src, dst, p, n)
    z = x
    for p in params["embed"][0]:
        z = _gat_sparse(z, src, dst, p, n)

    b_e = src // _NPER
    adj = jnp.zeros((_B, _NPER, _NPER), jnp.float32).at[
        b_e, src - b_e * _NPER, dst - b_e * _NPER].add(1.0)

    S = jax.nn.softmax(s.reshape(_B, _NPER, _K0), axis=-1)
    zd = z.reshape(_B, _NPER, -1)
    x1 = jnp.einsum("bnk,bnf->bkf", S, zd)

    ssq_adj = jnp.sum(adj * adj)
    a_s = jnp.einsum("bnm,bmk->bnk", adj, S)
    tr = jnp.sum(a_s * S)
    gram = jnp.einsum("bnk,bnl->bkl", S, S)
    gsq = jnp.sum(gram * gram)
    ll0 = jnp.sqrt(ssq_adj - 2.0 * tr + gsq) / (_B * _NPER * _NPER)
    el0 = jnp.mean(jnp.sum(-S * jnp.log(S + _EPS), axis=-1))

    h0 = params["heads"][0]
    c0 = jnp.mean(x1, axis=1) @ h0["W"] + h0["b"]

    z2 = x1
    for p in params["embed"][1]:
        z2 = _gat_dense(z2, p)
    x2 = jnp.sum(z2, axis=1)

    ll1 = jnp.float32(0.0)
    el1 = jnp.float32(0.0)

    h1 = params["heads"][1]
    out, c1 = _final_heads(x2, params["lin"]["W"], params["lin"]["b"],
                           h1["W"], h1["b"])
    return (out, ll0, ll1, el0, el1, c0, c1)


# speedup vs baseline: 7.7423x; 4.4635x over previous
import functools

import jax
import jax.numpy as jnp
from jax import lax
from jax.experimental import pallas as pl
from jax.experimental.pallas import tpu as pltpu
from jax.experimental.pallas import tpu_sc as plsc

_B = 10
_NPER = 1000
_K0 = 100
_NEG = 0.2
_EPS = 1e-15

_N = _B * _NPER
_E = 160000
_NW = 32
_REAL = _E // _NW
_EPW = 5120
_CH = 128
_NCH = _EPW // _CH
_NPC = 5000
_NPADC = 5120
_RPW = _NPADC // 16
_EPG = _E // _B


def _gat_sc_body(dp, xl_h, xr_h, src_h, dst_h, att_h, bias_h, out_h,
                 src_v, dst_v, dlbuf, rows_l, rows_r, ex_v, att_v,
                 bias_v, mrow, mgrid_v, m_all, s16f, shuf, den_loc, dstage,
                 outbuf, acc_s, den_s, mgrid_s):
    c = lax.axis_index("c")
    s = lax.axis_index("s")
    wid = c * 16 + s
    base_nodes = c * _NPC
    iota = lax.iota(jnp.int32, 16)
    zero16 = jnp.zeros((16,), jnp.float32)
    nt = dp // 16

    pltpu.sync_copy(src_h.at[pl.ds(wid * _EPW, _EPW)], src_v)
    pltpu.sync_copy(dst_h.at[pl.ds(wid * _EPW, _EPW)], dst_v)
    pltpu.sync_copy(att_h, att_v)
    pltpu.sync_copy(bias_h, bias_v)

    def _allreduce(v, op):
        for sh in (8, 4, 2, 1):
            shuf[pl.ds(0, 16)] = v
            v = op(v, plsc.load_gather(shuf, [iota ^ sh]))
        return v

    def _z1(i, _):
        def _z2(t, _):
            outbuf[i, pl.ds(t * 16, 16)] = zero16
            return 0
        return lax.fori_loop(0, nt, _z2, 0)
    lax.fori_loop(0, 64, _z1, 0)

    def _zc(b, _):
        pltpu.sync_copy(outbuf, acc_s.at[pl.ds(s * _RPW + b * 64, 64)])
        return 0
    lax.fori_loop(0, _RPW // 64, _zc, 0)

    def _chunk_a(k, m_loc):
        pltpu.sync_copy(xl_h.at[src_v.at[pl.ds(k * _CH, _CH)]], rows_l)
        pltpu.sync_copy(xr_h.at[dst_v.at[pl.ds(k * _CH, _CH)]], rows_r)

        def _grp(g, m_loc):
            p = k * _CH + g * 16

            def _edge(le, _):
                e = g * 16 + le

                def _t(t, acc):
                    u = (rows_l[e, pl.ds(t * 16, 16)]
                         + rows_r[e, pl.ds(t * 16, 16)])
                    lv = jnp.maximum(u, 0.0) + _NEG * jnp.minimum(u, 0.0)
                    return acc + att_v[pl.ds(t * 16, 16)] * lv

                acc = lax.fori_loop(0, nt, _t, zero16)
                s16f[pl.ds(le * 16, 16)] = acc
                return 0
            lax.fori_loop(0, 16, _edge, 0)
            logit = zero16
            for d in range(16):
                logit = logit + plsc.load_gather(s16f, [iota * 16 + d])
            pos = p + iota
            logit = jnp.where(pos < _REAL, logit, jnp.float32(-1e30))
            ex_v[pl.ds(p, 16)] = logit
            g0 = (wid * _REAL + p) // _EPG - c * 5
            g1 = (wid * _REAL + p + 15) // _EPG - c * 5
            glv = (wid * _REAL + pos) // _EPG - c * 5
            mx0 = _allreduce(jnp.where(glv == g0, logit,
                                       jnp.float32(-1e30)), jnp.maximum)
            mx1 = _allreduce(jnp.where(glv == g1, logit,
                                       jnp.float32(-1e30)), jnp.maximum)
            m_loc = jnp.where(iota == g0, jnp.maximum(m_loc, mx0), m_loc)
            m_loc = jnp.where(iota == g1, jnp.maximum(m_loc, mx1), m_loc)
            return m_loc

        return lax.fori_loop(0, _CH // 16, _grp, m_loc)

    m_loc = lax.fori_loop(0, _NCH, _chunk_a,
                          jnp.full((16,), -1e30, jnp.float32))

    mrow[pl.ds(0, 16)] = m_loc
    pltpu.sync_copy(mrow, mgrid_s.at[pl.ds(s * 16, 16)])
    plsc.subcore_barrier()
    pltpu.sync_copy(mgrid_s, mgrid_v)
    m_vec = mgrid_v[pl.ds(0, 16)]
    for w in range(1, 16):
        m_vec = jnp.maximum(m_vec, mgrid_v[pl.ds(w * 16, 16)])
    m_all[pl.ds(0, 16)] = m_vec

    def _zd(i, _):
        den_loc[pl.ds(i * 16, 16)] = zero16
        return 0
    lax.fori_loop(0, _NPADC // 16, _zd, 0)

    def _chunk_b(k, _):
        def _grp(g, _):
            p = k * _CH + g * 16
            pos = p + iota
            glv = (wid * _REAL + pos) // _EPG - c * 5
            mv = plsc.load_gather(m_all, [glv])
            lv = ex_v[pl.ds(p, 16)]
            ex = jnp.where(pos < _REAL, jnp.exp(lv - mv), 0.0)
            ex_v[pl.ds(p, 16)] = ex
            dl = dst_v[pl.ds(p, 16)] - base_nodes
            plsc.addupdate_scatter(den_loc, [dl], ex)
            return 0
        return lax.fori_loop(0, _CH // 16, _grp, 0)
    lax.fori_loop(0, _NCH, _chunk_b, 0)

    pltpu.sync_copy(den_loc, den_s.at[pl.ds(s * _NPADC, _NPADC)])
    plsc.subcore_barrier()

    def _chunk_c(k, _):
        pltpu.sync_copy(xl_h.at[src_v.at[pl.ds(k * _CH, _CH)]], rows_l)

        def _grp(g, _):
            p = k * _CH + g * 16
            dlbuf[pl.ds(g * 16, 16)] = dst_v[pl.ds(p, 16)] - base_nodes

            def _edge(le, _):
                e = g * 16 + le
                scv = plsc.load_gather(ex_v, [jnp.zeros((16,), jnp.int32)
                                              + (p + le)])

                def _t(t, _):
                    blk = rows_l[e, pl.ds(t * 16, 16)]
                    rows_l[e, pl.ds(t * 16, 16)] = blk * scv
                    return 0
                return lax.fori_loop(0, nt, _t, 0)
            return lax.fori_loop(0, 16, _edge, 0)
        lax.fori_loop(0, _CH // 16, _grp, 0)
        pltpu.sync_copy(rows_l, acc_s.at[dlbuf], add=True)
        return 0
    lax.fori_loop(0, _NCH, _chunk_c, 0)
    plsc.subcore_barrier()

    def _zd2(i, _):
        den_loc[pl.ds(i * 16, 16)] = zero16
        return 0
    lax.fori_loop(0, _RPW // 16, _zd2, 0)

    def _slot(w, _):
        pltpu.sync_copy(den_s.at[pl.ds(w * _NPADC + s * _RPW, _RPW)], dstage)

        def _add(i, _):
            den_loc[pl.ds(i * 16, 16)] = (den_loc[pl.ds(i * 16, 16)]
                                          + dstage[pl.ds(i * 16, 16)])
            return 0
        return lax.fori_loop(0, _RPW // 16, _add, 0)
    lax.fori_loop(0, 16, _slot, 0)

    def _blk(b, _):
        nb = s * _RPW + b * 64
        pltpu.sync_copy(acc_s.at[pl.ds(nb, 64)], outbuf)

        def _node(n, _):
            lnv = jnp.zeros((16,), jnp.int32) + (b * 64 + n)
            dv = plsc.load_gather(den_loc, [lnv])
            scv = 1.0 / jnp.maximum(dv, 1e-16)

            def _t(t, _):
                outbuf[n, pl.ds(t * 16, 16)] = (
                    outbuf[n, pl.ds(t * 16, 16)] * scv
                    + bias_v[pl.ds(t * 16, 16)])
                return 0
            return lax.fori_loop(0, nt, _t, 0)
        lax.fori_loop(0, 64, _node, 0)
        pltpu.sync_copy(outbuf, out_h.at[c, pl.ds(nb, 64)])
        return 0
    lax.fori_loop(0, _RPW // 64, _blk, 0)


@functools.cache
def _make_gat_sc(dp):
    mesh = plsc.VectorSubcoreMesh(core_axis_name="c", subcore_axis_name="s",
                                  num_cores=2, num_subcores=16)
    return functools.partial(
        pl.kernel,
        out_type=jax.ShapeDtypeStruct((2, _NPADC, dp), jnp.float32),
        mesh=mesh,
        compiler_params=pltpu.CompilerParams(needs_layout_passes=False),
        scratch_types=[
            pltpu.VMEM((_EPW,), jnp.int32),
            pltpu.VMEM((_EPW,), jnp.int32),
            pltpu.VMEM((_CH,), jnp.int32),
            pltpu.VMEM((_CH, dp), jnp.float32),
            pltpu.VMEM((_CH, dp), jnp.float32),
            pltpu.VMEM((_EPW,), jnp.float32),
            pltpu.VMEM((dp,), jnp.float32),
            pltpu.VMEM((dp,), jnp.float32),
            pltpu.VMEM((16,), jnp.float32),
            pltpu.VMEM((256,), jnp.float32),
            pltpu.VMEM((128,), jnp.float32),
            pltpu.VMEM((256,), jnp.float32),
            pltpu.VMEM((128,), jnp.float32),
            pltpu.VMEM((_NPADC,), jnp.float32),
            pltpu.VMEM((_RPW,), jnp.float32),
            pltpu.VMEM((64, dp), jnp.float32),
            pltpu.VMEM_SHARED((_NPADC, dp), jnp.float32),
            pltpu.VMEM_SHARED((16 * _NPADC,), jnp.float32),
            pltpu.VMEM_SHARED((256,), jnp.float32),
        ],
    )(functools.partial(_gat_sc_body, dp))


def _proj_body(x_ref, w_ref, b_ref, o_ref):
    o_ref[...] = jnp.dot(x_ref[...], w_ref[...],
                         preferred_element_type=jnp.float32) + b_ref[...]


def _proj(x, w, b):
    n, din = x.shape
    dout = w.shape[1]
    blk = 1000
    return pl.pallas_call(
        _proj_body,
        grid=(n // blk,),
        in_specs=[pl.BlockSpec((blk, din), lambda i: (i, 0)),
                  pl.BlockSpec((din, dout), lambda i: (0, 0)),
                  pl.BlockSpec((1, dout), lambda i: (0, 0))],
        out_specs=pl.BlockSpec((blk, dout), lambda i: (i, 0)),
        out_shape=jax.ShapeDtypeStruct((n, dout), jnp.float32),
    )(x, w, b.reshape(1, -1))


def _pad_cols(a, dp):
    if a.shape[-1] == dp:
        return a
    return jnp.pad(a, [(0, 0)] * (a.ndim - 1) + [(0, dp - a.shape[-1])])


def _gat_sparse_sc(x, srcp, dstp, p, dout):
    dp = 128
    w2 = jnp.concatenate([_pad_cols(p["Wl"], dp), _pad_cols(p["Wr"], dp)],
                         axis=1)
    b2 = jnp.concatenate([_pad_cols(p["bl"], dp), _pad_cols(p["br"], dp)])
    xlr = _proj(x, w2, b2)
    xl, xr = xlr[:, :dp], xlr[:, dp:]
    att = _pad_cols(p["att"], dp)
    bias = _pad_cols(p["bias"], dp)
    out2 = _make_gat_sc(dp)(xl, xr, srcp, dstp, att, bias)
    out = jnp.concatenate([out2[0, :_NPC], out2[1, :_NPC]], axis=0)
    return out[:, :dout]


def _gat_dense(z, p):
    xl = z @ p["Wl"] + p["bl"]
    xr = z @ p["Wr"] + p["br"]
    e = xl[:, None, :, :] + xr[:, :, None, :]
    e = jnp.where(e >= 0, e, _NEG * e)
    logits = jnp.einsum("bijd,d->bij", e, p["att"])
    alpha = jax.nn.softmax(logits, axis=-1)
    return jnp.einsum("bij,bjd->bid", alpha, xl) + p["bias"]


def _heads_body(x_ref, w_ref, b_ref, o_ref):
    o_ref[...] = jnp.dot(x_ref[...], w_ref[...],
                         preferred_element_type=jnp.float32) + b_ref[...]


def _final_heads(x2, w_lin, b_lin, w_h1, b_h1):
    xp = jnp.zeros((16, 128), jnp.float32).at[:_B, :].set(x2)
    wp = jnp.zeros((128, 128), jnp.float32)
    wp = wp.at[:, 0:2].set(w_lin).at[:, 2:4].set(w_h1)
    bp = jnp.zeros((1, 128), jnp.float32)
    bp = bp.at[0, 0:2].set(b_lin).at[0, 2:4].set(b_h1)
    o = pl.pallas_call(
        _heads_body,
        out_shape=jax.ShapeDtypeStruct((16, 128), jnp.float32),
    )(xp, wp, bp)
    return o[:_B, 0:2], o[:_B, 2:4]


def kernel(x, edge_index, batch, params):
    del batch
    src, dst = edge_index[0], edge_index[1]

    fill = jnp.repeat(jnp.array([0, _NPC], jnp.int32), 16)[:, None]
    col_ok = jnp.arange(_EPW, dtype=jnp.int32)[None, :] < _REAL
    srcp = jnp.where(col_ok, jnp.pad(src.reshape(_NW, _REAL),
                                     ((0, 0), (0, _EPW - _REAL))),
                     fill).reshape(-1)
    dstp = jnp.where(col_ok, jnp.pad(dst.reshape(_NW, _REAL),
                                     ((0, 0), (0, _EPW - _REAL))),
                     fill).reshape(-1)

    s = x
    for p, dout in zip(params["pool"][0], (64, 100)):
        s = _gat_sparse_sc(s, srcp, dstp, p, dout)
    z = x
    for p, dout in zip(params["embed"][0], (128, 128)):
        z = _gat_sparse_sc(z, srcp, dstp, p, dout)

    b_e = src // _NPER
    adj = jnp.zeros((_B, _NPER, _NPER), jnp.float32).at[
        b_e, src - b_e * _NPER, dst - b_e * _NPER].add(1.0)

    S = jax.nn.softmax(s.reshape(_B, _NPER, _K0), axis=-1)
    zd = z.reshape(_B, _NPER, -1)
    x1 = jnp.einsum("bnk,bnf->bkf", S, zd)

    ssq_adj = jnp.sum(adj * adj)
    a_s = jnp.einsum("bnm,bmk->bnk", adj, S)
    tr = jnp.sum(a_s * S)
    gram = jnp.einsum("bnk,bnl->bkl", S, S)
    gsq = jnp.sum(gram * gram)
    ll0 = jnp.sqrt(ssq_adj - 2.0 * tr + gsq) / (_B * _NPER * _NPER)
    el0 = jnp.mean(jnp.sum(-S * jnp.log(S + _EPS), axis=-1))

    h0 = params["heads"][0]
    c0 = jnp.mean(x1, axis=1) @ h0["W"] + h0["b"]

    z2 = x1
    for p in params["embed"][1]:
        z2 = _gat_dense(z2, p)
    x2 = jnp.sum(z2, axis=1)

    ll1 = jnp.float32(0.0)
    el1 = jnp.float32(0.0)

    h1 = params["heads"][1]
    out, c1 = _final_heads(x2, params["lin"]["W"], params["lin"]["b"],
                           h1["W"], h1["b"])
    return (out, ll0, ll1, el0, el1, c0, c1)


# speedup vs baseline: 7.8311x; 1.0115x over previous
import functools

import jax
import jax.numpy as jnp
from jax import lax
from jax.experimental import pallas as pl
from jax.experimental.pallas import tpu as pltpu
from jax.experimental.pallas import tpu_sc as plsc

_B = 10
_NPER = 1000
_K0 = 100
_NEG = 0.2
_EPS = 1e-15

_N = _B * _NPER
_E = 160000
_NW = 32
_REAL = _E // _NW
_EPW = 5120
_CH = 128
_NCH = _EPW // _CH
_NPC = 5000
_NPADC = 5120
_RPW = _NPADC // 16
_EPG = _E // _B


def _gat_sc_body(dp, xl_h, xr_h, src_h, dst_h, att_h, bias_h, out_h,
                 src_v, dst_v, dlbuf, rows_l, rows_r, ex_v, att_v,
                 bias_v, mrow, mgrid_v, m_all, s16f, shuf, den_loc, dstage,
                 outbuf, acc_s, den_s, mgrid_s):
    c = lax.axis_index("c")
    s = lax.axis_index("s")
    wid = c * 16 + s
    base_nodes = c * _NPC
    iota = lax.iota(jnp.int32, 16)
    zero16 = jnp.zeros((16,), jnp.float32)
    nt = dp // 16

    pltpu.sync_copy(src_h.at[pl.ds(wid * _EPW, _EPW)], src_v)
    pltpu.sync_copy(dst_h.at[pl.ds(wid * _EPW, _EPW)], dst_v)
    pltpu.sync_copy(att_h, att_v)
    pltpu.sync_copy(bias_h, bias_v)

    def _allreduce(v, op):
        for sh in (8, 4, 2, 1):
            shuf[pl.ds(0, 16)] = v
            v = op(v, plsc.load_gather(shuf, [iota ^ sh]))
        return v

    def _z1(i, _):
        def _z2(t, _):
            outbuf[i, pl.ds(t * 16, 16)] = zero16
            return 0
        return lax.fori_loop(0, nt, _z2, 0)
    lax.fori_loop(0, 64, _z1, 0)

    def _zc(b, _):
        pltpu.sync_copy(outbuf, acc_s.at[pl.ds(s * _RPW + b * 64, 64)])
        return 0
    lax.fori_loop(0, _RPW // 64, _zc, 0)

    def _chunk_a(k, m_loc):
        pltpu.sync_copy(xl_h.at[src_v.at[pl.ds(k * _CH, _CH)]], rows_l)
        pltpu.sync_copy(xr_h.at[dst_v.at[pl.ds(k * _CH, _CH)]], rows_r)

        def _grp(g, m_loc):
            p = k * _CH + g * 16

            def _edge(le, _):
                e = g * 16 + le
                acc = zero16
                for t in range(nt):
                    u = (rows_l[e, pl.ds(t * 16, 16)]
                         + rows_r[e, pl.ds(t * 16, 16)])
                    lv = jnp.maximum(u, _NEG * u)
                    acc = acc + att_v[pl.ds(t * 16, 16)] * lv
                s16f[pl.ds(le * 16, 16)] = acc
                return 0
            lax.fori_loop(0, 16, _edge, 0)
            logit = zero16
            for d in range(16):
                logit = logit + plsc.load_gather(s16f, [iota * 16 + d])
            pos = p + iota
            logit = jnp.where(pos < _REAL, logit, jnp.float32(-1e30))
            ex_v[pl.ds(p, 16)] = logit
            g0 = (wid * _REAL + p) // _EPG - c * 5
            g1 = (wid * _REAL + p + 15) // _EPG - c * 5
            glv = (wid * _REAL + pos) // _EPG - c * 5
            mx0 = _allreduce(jnp.where(glv == g0, logit,
                                       jnp.float32(-1e30)), jnp.maximum)
            mx1 = _allreduce(jnp.where(glv == g1, logit,
                                       jnp.float32(-1e30)), jnp.maximum)
            m_loc = jnp.where(iota == g0, jnp.maximum(m_loc, mx0), m_loc)
            m_loc = jnp.where(iota == g1, jnp.maximum(m_loc, mx1), m_loc)
            return m_loc

        return lax.fori_loop(0, _CH // 16, _grp, m_loc)

    m_loc = lax.fori_loop(0, _NCH, _chunk_a,
                          jnp.full((16,), -1e30, jnp.float32))

    mrow[pl.ds(0, 16)] = m_loc
    pltpu.sync_copy(mrow, mgrid_s.at[pl.ds(s * 16, 16)])
    plsc.subcore_barrier()
    pltpu.sync_copy(mgrid_s, mgrid_v)
    m_vec = mgrid_v[pl.ds(0, 16)]
    for w in range(1, 16):
        m_vec = jnp.maximum(m_vec, mgrid_v[pl.ds(w * 16, 16)])
    m_all[pl.ds(0, 16)] = m_vec

    def _zd(i, _):
        den_loc[pl.ds(i * 16, 16)] = zero16
        return 0
    lax.fori_loop(0, _NPADC // 16, _zd, 0)

    def _chunk_b(k, _):
        def _grp(g, _):
            p = k * _CH + g * 16
            pos = p + iota
            glv = (wid * _REAL + pos) // _EPG - c * 5
            mv = plsc.load_gather(m_all, [glv])
            lv = ex_v[pl.ds(p, 16)]
            ex = jnp.where(pos < _REAL, jnp.exp(lv - mv), 0.0)
            ex_v[pl.ds(p, 16)] = ex
            dl = dst_v[pl.ds(p, 16)] - base_nodes
            plsc.addupdate_scatter(den_loc, [dl], ex)
            return 0
        return lax.fori_loop(0, _CH // 16, _grp, 0)
    lax.fori_loop(0, _NCH, _chunk_b, 0)

    pltpu.sync_copy(den_loc, den_s.at[pl.ds(s * _NPADC, _NPADC)])
    plsc.subcore_barrier()

    def _chunk_c(k, _):
        pltpu.sync_copy(xl_h.at[src_v.at[pl.ds(k * _CH, _CH)]], rows_l)

        def _grp(g, _):
            p = k * _CH + g * 16
            dlbuf[pl.ds(g * 16, 16)] = dst_v[pl.ds(p, 16)] - base_nodes

            def _edge(le, _):
                e = g * 16 + le
                scv = plsc.load_gather(ex_v, [jnp.zeros((16,), jnp.int32)
                                              + (p + le)])

                for t in range(nt):
                    blk = rows_l[e, pl.ds(t * 16, 16)]
                    rows_l[e, pl.ds(t * 16, 16)] = blk * scv
                return 0
            return lax.fori_loop(0, 16, _edge, 0)
        lax.fori_loop(0, _CH // 16, _grp, 0)
        pltpu.sync_copy(rows_l, acc_s.at[dlbuf], add=True)
        return 0
    lax.fori_loop(0, _NCH, _chunk_c, 0)
    plsc.subcore_barrier()

    def _zd2(i, _):
        den_loc[pl.ds(i * 16, 16)] = zero16
        return 0
    lax.fori_loop(0, _RPW // 16, _zd2, 0)

    def _slot(w, _):
        pltpu.sync_copy(den_s.at[pl.ds(w * _NPADC + s * _RPW, _RPW)], dstage)

        def _add(i, _):
            den_loc[pl.ds(i * 16, 16)] = (den_loc[pl.ds(i * 16, 16)]
                                          + dstage[pl.ds(i * 16, 16)])
            return 0
        return lax.fori_loop(0, _RPW // 16, _add, 0)
    lax.fori_loop(0, 16, _slot, 0)

    def _blk(b, _):
        nb = s * _RPW + b * 64
        pltpu.sync_copy(acc_s.at[pl.ds(nb, 64)], outbuf)

        def _node(n, _):
            lnv = jnp.zeros((16,), jnp.int32) + (b * 64 + n)
            dv = plsc.load_gather(den_loc, [lnv])
            scv = 1.0 / jnp.maximum(dv, 1e-16)

            for t in range(nt):
                outbuf[n, pl.ds(t * 16, 16)] = (
                    outbuf[n, pl.ds(t * 16, 16)] * scv
                    + bias_v[pl.ds(t * 16, 16)])
            return 0
        lax.fori_loop(0, 64, _node, 0)
        pltpu.sync_copy(outbuf, out_h.at[c, pl.ds(nb, 64)])
        return 0
    lax.fori_loop(0, _RPW // 64, _blk, 0)


@functools.cache
def _make_gat_sc(dp):
    mesh = plsc.VectorSubcoreMesh(core_axis_name="c", subcore_axis_name="s",
                                  num_cores=2, num_subcores=16)
    return functools.partial(
        pl.kernel,
        out_type=jax.ShapeDtypeStruct((2, _NPADC, dp), jnp.float32),
        mesh=mesh,
        compiler_params=pltpu.CompilerParams(needs_layout_passes=False),
        scratch_types=[
            pltpu.VMEM((_EPW,), jnp.int32),
            pltpu.VMEM((_EPW,), jnp.int32),
            pltpu.VMEM((_CH,), jnp.int32),
            pltpu.VMEM((_CH, dp), jnp.float32),
            pltpu.VMEM((_CH, dp), jnp.float32),
            pltpu.VMEM((_EPW,), jnp.float32),
            pltpu.VMEM((dp,), jnp.float32),
            pltpu.VMEM((dp,), jnp.float32),
            pltpu.VMEM((16,), jnp.float32),
            pltpu.VMEM((256,), jnp.float32),
            pltpu.VMEM((128,), jnp.float32),
            pltpu.VMEM((256,), jnp.float32),
            pltpu.VMEM((128,), jnp.float32),
            pltpu.VMEM((_NPADC,), jnp.float32),
            pltpu.VMEM((_RPW,), jnp.float32),
            pltpu.VMEM((64, dp), jnp.float32),
            pltpu.VMEM_SHARED((_NPADC, dp), jnp.float32),
            pltpu.VMEM_SHARED((16 * _NPADC,), jnp.float32),
            pltpu.VMEM_SHARED((256,), jnp.float32),
        ],
    )(functools.partial(_gat_sc_body, dp))


def _proj_body(x_ref, w_ref, b_ref, o_ref):
    o_ref[...] = jnp.dot(x_ref[...], w_ref[...],
                         preferred_element_type=jnp.float32) + b_ref[...]


def _proj(x, w, b):
    n, din = x.shape
    dout = w.shape[1]
    blk = 1000
    return pl.pallas_call(
        _proj_body,
        grid=(n // blk,),
        in_specs=[pl.BlockSpec((blk, din), lambda i: (i, 0)),
                  pl.BlockSpec((din, dout), lambda i: (0, 0)),
                  pl.BlockSpec((1, dout), lambda i: (0, 0))],
        out_specs=pl.BlockSpec((blk, dout), lambda i: (i, 0)),
        out_shape=jax.ShapeDtypeStruct((n, dout), jnp.float32),
    )(x, w, b.reshape(1, -1))


def _pad_cols(a, dp):
    if a.shape[-1] == dp:
        return a
    return jnp.pad(a, [(0, 0)] * (a.ndim - 1) + [(0, dp - a.shape[-1])])


def _gat_sparse_sc(x, srcp, dstp, p, dout):
    dp = 128
    w2 = jnp.concatenate([_pad_cols(p["Wl"], dp), _pad_cols(p["Wr"], dp)],
                         axis=1)
    b2 = jnp.concatenate([_pad_cols(p["bl"], dp), _pad_cols(p["br"], dp)])
    xlr = _proj(x, w2, b2)
    xl, xr = xlr[:, :dp], xlr[:, dp:]
    att = _pad_cols(p["att"], dp)
    bias = _pad_cols(p["bias"], dp)
    out2 = _make_gat_sc(dp)(xl, xr, srcp, dstp, att, bias)
    out = jnp.concatenate([out2[0, :_NPC], out2[1, :_NPC]], axis=0)
    return out[:, :dout]


def _gat_dense(z, p):
    xl = z @ p["Wl"] + p["bl"]
    xr = z @ p["Wr"] + p["br"]
    e = xl[:, None, :, :] + xr[:, :, None, :]
    e = jnp.where(e >= 0, e, _NEG * e)
    logits = jnp.einsum("bijd,d->bij", e, p["att"])
    alpha = jax.nn.softmax(logits, axis=-1)
    return jnp.einsum("bij,bjd->bid", alpha, xl) + p["bias"]


def _heads_body(x_ref, w_ref, b_ref, o_ref):
    o_ref[...] = jnp.dot(x_ref[...], w_ref[...],
                         preferred_element_type=jnp.float32) + b_ref[...]


def _final_heads(x2, w_lin, b_lin, w_h1, b_h1):
    xp = jnp.zeros((16, 128), jnp.float32).at[:_B, :].set(x2)
    wp = jnp.zeros((128, 128), jnp.float32)
    wp = wp.at[:, 0:2].set(w_lin).at[:, 2:4].set(w_h1)
    bp = jnp.zeros((1, 128), jnp.float32)
    bp = bp.at[0, 0:2].set(b_lin).at[0, 2:4].set(b_h1)
    o = pl.pallas_call(
        _heads_body,
        out_shape=jax.ShapeDtypeStruct((16, 128), jnp.float32),
    )(xp, wp, bp)
    return o[:_B, 0:2], o[:_B, 2:4]


def kernel(x, edge_index, batch, params):
    del batch
    src, dst = edge_index[0], edge_index[1]

    fill = jnp.repeat(jnp.array([0, _NPC], jnp.int32), 16)[:, None]
    col_ok = jnp.arange(_EPW, dtype=jnp.int32)[None, :] < _REAL
    srcp = jnp.where(col_ok, jnp.pad(src.reshape(_NW, _REAL),
                                     ((0, 0), (0, _EPW - _REAL))),
                     fill).reshape(-1)
    dstp = jnp.where(col_ok, jnp.pad(dst.reshape(_NW, _REAL),
                                     ((0, 0), (0, _EPW - _REAL))),
                     fill).reshape(-1)

    s = x
    for p, dout in zip(params["pool"][0], (64, 100)):
        s = _gat_sparse_sc(s, srcp, dstp, p, dout)
    z = x
    for p, dout in zip(params["embed"][0], (128, 128)):
        z = _gat_sparse_sc(z, srcp, dstp, p, dout)

    b_e = src // _NPER
    adj = jnp.zeros((_B, _NPER, _NPER), jnp.float32).at[
        b_e, src - b_e * _NPER, dst - b_e * _NPER].add(1.0)

    S = jax.nn.softmax(s.reshape(_B, _NPER, _K0), axis=-1)
    zd = z.reshape(_B, _NPER, -1)
    x1 = jnp.einsum("bnk,bnf->bkf", S, zd)

    ssq_adj = jnp.sum(adj * adj)
    a_s = jnp.einsum("bnm,bmk->bnk", adj, S)
    tr = jnp.sum(a_s * S)
    gram = jnp.einsum("bnk,bnl->bkl", S, S)
    gsq = jnp.sum(gram * gram)
    ll0 = jnp.sqrt(ssq_adj - 2.0 * tr + gsq) / (_B * _NPER * _NPER)
    el0 = jnp.mean(jnp.sum(-S * jnp.log(S + _EPS), axis=-1))

    h0 = params["heads"][0]
    c0 = jnp.mean(x1, axis=1) @ h0["W"] + h0["b"]

    z2 = x1
    for p in params["embed"][1]:
        z2 = _gat_dense(z2, p)
    x2 = jnp.sum(z2, axis=1)

    ll1 = jnp.float32(0.0)
    el1 = jnp.float32(0.0)

    h1 = params["heads"][1]
    out, c1 = _final_heads(x2, params["lin"]["W"], params["lin"]["b"],
                           h1["W"], h1["b"])
    return (out, ll0, ll1, el0, el1, c0, c1)


# speedup vs baseline: 11.9122x; 1.5211x over previous
import functools

import jax
import jax.numpy as jnp
from jax import lax
from jax.experimental import pallas as pl
from jax.experimental.pallas import tpu as pltpu
from jax.experimental.pallas import tpu_sc as plsc

_B = 10
_NPER = 1000
_K0 = 100
_NEG = 0.2
_EPS = 1e-15

_N = _B * _NPER
_E = 160000
_NW = 32
_REAL = _E // _NW
_EPW = 5120
_CH = 128
_NCH = _EPW // _CH
_NPC = 5000
_NPADC = 5120
_RPW = _NPADC // 16
_EPG = _E // _B


def _gat_sc_body(dp, xl_h, xr_h, src_h, dst_h, att_h, bias_h, out_h,
                 src_v, dst_v, dlb_a, dlb_b, l0, r0, l1, r1, exbuf, att_v,
                 bias_v, s16f, den_loc, dstage, outbuf,
                 g0l, g0r, g1l, g1r, ss0, ss1,
                 acc_s, den_s):
    c = lax.axis_index("c")
    s = lax.axis_index("s")
    wid = c * 16 + s
    base_nodes = c * _NPC
    iota = lax.iota(jnp.int32, 16)
    zero16 = jnp.zeros((16,), jnp.float32)
    nt = dp // 16

    pltpu.sync_copy(src_h.at[pl.ds(wid * _EPW, _EPW)], src_v)
    pltpu.sync_copy(dst_h.at[pl.ds(wid * _EPW, _EPW)], dst_v)
    pltpu.sync_copy(att_h, att_v)
    pltpu.sync_copy(bias_h, bias_v)

    def _z1(i, _):
        for t in range(nt):
            outbuf[i, pl.ds(t * 16, 16)] = zero16
        return 0
    lax.fori_loop(0, 16, _z1, 0)

    def _zc(b, _):
        pltpu.sync_copy(outbuf, acc_s.at[pl.ds(s * _RPW + b * 16, 16)])
        return 0
    lax.fori_loop(0, _RPW // 16, _zc, 0)

    def _zd(i, _):
        den_loc[pl.ds(i * 16, 16)] = zero16
        return 0
    lax.fori_loop(0, _NPADC // 16, _zd, 0)

    def _half(k, lbuf, rbuf, dlb):
        def _grp(g, _):
            p = k * _CH + g * 16

            def _edge(le, _):
                e = g * 16 + le
                acc = zero16
                for t in range(nt):
                    u = lbuf[e, pl.ds(t * 16, 16)] + rbuf[e, pl.ds(t * 16, 16)]
                    acc = acc + (att_v[pl.ds(t * 16, 16)]
                                 * jnp.maximum(u, _NEG * u))
                s16f[pl.ds(le * 16, 16)] = acc
                return 0
            lax.fori_loop(0, 16, _edge, 0)
            logit = zero16
            for d in range(16):
                logit = logit + plsc.load_gather(s16f, [iota * 16 + d])
            pos = p + iota
            ex = jnp.where(pos < _REAL, jnp.exp(logit), 0.0)
            exbuf[pl.ds(g * 16, 16)] = ex
            dl = dst_v[pl.ds(p, 16)] - base_nodes
            dlb[pl.ds(g * 16, 16)] = dl
            plsc.addupdate_scatter(den_loc, [dl], ex)

            def _sc(le, _):
                e = g * 16 + le
                scv = plsc.load_gather(exbuf,
                                       [jnp.zeros((16,), jnp.int32) + e])
                for t in range(nt):
                    blk = lbuf[e, pl.ds(t * 16, 16)]
                    lbuf[e, pl.ds(t * 16, 16)] = blk * scv
                return 0
            return lax.fori_loop(0, 16, _sc, 0)
        lax.fori_loop(0, _CH // 16, _grp, 0)

    def _round(j, _):
        a = 2 * j
        b = 2 * j + 1
        da_l = pltpu.async_copy(xl_h.at[src_v.at[pl.ds(a * _CH, _CH)]],
                                l0, g0l)
        da_r = pltpu.async_copy(xr_h.at[dst_v.at[pl.ds(a * _CH, _CH)]],
                                r0, g0r)
        db_l = pltpu.async_copy(xl_h.at[src_v.at[pl.ds(b * _CH, _CH)]],
                                l1, g1l)
        db_r = pltpu.async_copy(xr_h.at[dst_v.at[pl.ds(b * _CH, _CH)]],
                                r1, g1r)
        da_l.wait()
        da_r.wait()
        _half(a, l0, r0, dlb_a)
        dsa = pltpu.async_copy(l0, acc_s.at[dlb_a], ss0, add=True)
        db_l.wait()
        db_r.wait()
        _half(b, l1, r1, dlb_b)
        dsb = pltpu.async_copy(l1, acc_s.at[dlb_b], ss1, add=True)
        dsa.wait()
        dsb.wait()
        return 0
    lax.fori_loop(0, _NCH // 2, _round, 0)

    pltpu.sync_copy(den_loc, den_s.at[pl.ds(s * _NPADC, _NPADC)])
    plsc.subcore_barrier()

    def _zd2(i, _):
        den_loc[pl.ds(i * 16, 16)] = zero16
        return 0
    lax.fori_loop(0, _RPW // 16, _zd2, 0)

    def _slot(w, _):
        pltpu.sync_copy(den_s.at[pl.ds(w * _NPADC + s * _RPW, _RPW)], dstage)

        def _add(i, _):
            den_loc[pl.ds(i * 16, 16)] = (den_loc[pl.ds(i * 16, 16)]
                                          + dstage[pl.ds(i * 16, 16)])
            return 0
        return lax.fori_loop(0, _RPW // 16, _add, 0)
    lax.fori_loop(0, 16, _slot, 0)

    def _blk(b, _):
        nb = s * _RPW + b * 16
        pltpu.sync_copy(acc_s.at[pl.ds(nb, 16)], outbuf)

        def _node(n, _):
            lnv = jnp.zeros((16,), jnp.int32) + (b * 16 + n)
            dv = plsc.load_gather(den_loc, [lnv])
            scv = 1.0 / jnp.maximum(dv, 1e-16)
            for t in range(nt):
                outbuf[n, pl.ds(t * 16, 16)] = (
                    outbuf[n, pl.ds(t * 16, 16)] * scv
                    + bias_v[pl.ds(t * 16, 16)])
            return 0
        lax.fori_loop(0, 16, _node, 0)
        pltpu.sync_copy(outbuf, out_h.at[c, pl.ds(nb, 16)])
        return 0
    lax.fori_loop(0, _RPW // 16, _blk, 0)


@functools.cache
def _make_gat_sc(dp):
    mesh = plsc.VectorSubcoreMesh(core_axis_name="c", subcore_axis_name="s",
                                  num_cores=2, num_subcores=16)
    return functools.partial(
        pl.kernel,
        out_type=jax.ShapeDtypeStruct((2, _NPADC, dp), jnp.float32),
        mesh=mesh,
        compiler_params=pltpu.CompilerParams(needs_layout_passes=False),
        scratch_types=[
            pltpu.VMEM((_EPW,), jnp.int32),
            pltpu.VMEM((_EPW,), jnp.int32),
            pltpu.VMEM((_CH,), jnp.int32),
            pltpu.VMEM((_CH,), jnp.int32),
            pltpu.VMEM((_CH, dp), jnp.float32),
            pltpu.VMEM((_CH, dp), jnp.float32),
            pltpu.VMEM((_CH, dp), jnp.float32),
            pltpu.VMEM((_CH, dp), jnp.float32),
            pltpu.VMEM((_CH,), jnp.float32),
            pltpu.VMEM((dp,), jnp.float32),
            pltpu.VMEM((dp,), jnp.float32),
            pltpu.VMEM((256,), jnp.float32),
            pltpu.VMEM((_NPADC,), jnp.float32),
            pltpu.VMEM((_RPW,), jnp.float32),
            pltpu.VMEM((16, dp), jnp.float32),
            pltpu.SemaphoreType.DMA,
            pltpu.SemaphoreType.DMA,
            pltpu.SemaphoreType.DMA,
            pltpu.SemaphoreType.DMA,
            pltpu.SemaphoreType.DMA,
            pltpu.SemaphoreType.DMA,
            pltpu.VMEM_SHARED((_NPADC, dp), jnp.float32),
            pltpu.VMEM_SHARED((16 * _NPADC,), jnp.float32),
        ],
    )(functools.partial(_gat_sc_body, dp))


def _proj_body(x_ref, w_ref, b_ref, o_ref):
    o_ref[...] = jnp.dot(x_ref[...], w_ref[...],
                         preferred_element_type=jnp.float32) + b_ref[...]


def _proj(x, w, b):
    n, din = x.shape
    dout = w.shape[1]
    blk = 1000
    return pl.pallas_call(
        _proj_body,
        grid=(n // blk,),
        in_specs=[pl.BlockSpec((blk, din), lambda i: (i, 0)),
                  pl.BlockSpec((din, dout), lambda i: (0, 0)),
                  pl.BlockSpec((1, dout), lambda i: (0, 0))],
        out_specs=pl.BlockSpec((blk, dout), lambda i: (i, 0)),
        out_shape=jax.ShapeDtypeStruct((n, dout), jnp.float32),
    )(x, w, b.reshape(1, -1))


def _pad_cols(a, dp):
    if a.shape[-1] == dp:
        return a
    return jnp.pad(a, [(0, 0)] * (a.ndim - 1) + [(0, dp - a.shape[-1])])


def _gat_sparse_sc(x, srcp, dstp, p, dout):
    dp = 128
    w2 = jnp.concatenate([_pad_cols(p["Wl"], dp), _pad_cols(p["Wr"], dp)],
                         axis=1)
    b2 = jnp.concatenate([_pad_cols(p["bl"], dp), _pad_cols(p["br"], dp)])
    xlr = _proj(x, w2, b2)
    xl, xr = xlr[:, :dp], xlr[:, dp:]
    att = _pad_cols(p["att"], dp)
    bias = _pad_cols(p["bias"], dp)
    out2 = _make_gat_sc(dp)(xl, xr, srcp, dstp, att, bias)
    out = jnp.concatenate([out2[0, :_NPC], out2[1, :_NPC]], axis=0)
    return out[:, :dout]


def _gat_dense(z, p):
    xl = z @ p["Wl"] + p["bl"]
    xr = z @ p["Wr"] + p["br"]
    e = xl[:, None, :, :] + xr[:, :, None, :]
    e = jnp.where(e >= 0, e, _NEG * e)
    logits = jnp.einsum("bijd,d->bij", e, p["att"])
    alpha = jax.nn.softmax(logits, axis=-1)
    return jnp.einsum("bij,bjd->bid", alpha, xl) + p["bias"]


def _heads_body(x_ref, w_ref, b_ref, o_ref):
    o_ref[...] = jnp.dot(x_ref[...], w_ref[...],
                         preferred_element_type=jnp.float32) + b_ref[...]


def _final_heads(x2, w_lin, b_lin, w_h1, b_h1):
    xp = jnp.zeros((16, 128), jnp.float32).at[:_B, :].set(x2)
    wp = jnp.zeros((128, 128), jnp.float32)
    wp = wp.at[:, 0:2].set(w_lin).at[:, 2:4].set(w_h1)
    bp = jnp.zeros((1, 128), jnp.float32)
    bp = bp.at[0, 0:2].set(b_lin).at[0, 2:4].set(b_h1)
    o = pl.pallas_call(
        _heads_body,
        out_shape=jax.ShapeDtypeStruct((16, 128), jnp.float32),
    )(xp, wp, bp)
    return o[:_B, 0:2], o[:_B, 2:4]


def kernel(x, edge_index, batch, params):
    del batch
    src, dst = edge_index[0], edge_index[1]

    fill = jnp.repeat(jnp.array([0, _NPC], jnp.int32), 16)[:, None]
    col_ok = jnp.arange(_EPW, dtype=jnp.int32)[None, :] < _REAL
    srcp = jnp.where(col_ok, jnp.pad(src.reshape(_NW, _REAL),
                                     ((0, 0), (0, _EPW - _REAL))),
                     fill).reshape(-1)
    dstp = jnp.where(col_ok, jnp.pad(dst.reshape(_NW, _REAL),
                                     ((0, 0), (0, _EPW - _REAL))),
                     fill).reshape(-1)

    s = x
    for p, dout in zip(params["pool"][0], (64, 100)):
        s = _gat_sparse_sc(s, srcp, dstp, p, dout)
    z = x
    for p, dout in zip(params["embed"][0], (128, 128)):
        z = _gat_sparse_sc(z, srcp, dstp, p, dout)

    b_e = src // _NPER
    adj = jnp.zeros((_B, _NPER, _NPER), jnp.float32).at[
        b_e, src - b_e * _NPER, dst - b_e * _NPER].add(1.0)

    S = jax.nn.softmax(s.reshape(_B, _NPER, _K0), axis=-1)
    zd = z.reshape(_B, _NPER, -1)
    x1 = jnp.einsum("bnk,bnf->bkf", S, zd)

    ssq_adj = jnp.sum(adj * adj)
    a_s = jnp.einsum("bnm,bmk->bnk", adj, S)
    tr = jnp.sum(a_s * S)
    gram = jnp.einsum("bnk,bnl->bkl", S, S)
    gsq = jnp.sum(gram * gram)
    ll0 = jnp.sqrt(ssq_adj - 2.0 * tr + gsq) / (_B * _NPER * _NPER)
    el0 = jnp.mean(jnp.sum(-S * jnp.log(S + _EPS), axis=-1))

    h0 = params["heads"][0]
    c0 = jnp.mean(x1, axis=1) @ h0["W"] + h0["b"]

    z2 = x1
    for p in params["embed"][1]:
        z2 = _gat_dense(z2, p)
    x2 = jnp.sum(z2, axis=1)

    ll1 = jnp.float32(0.0)
    el1 = jnp.float32(0.0)

    h1 = params["heads"][1]
    out, c1 = _final_heads(x2, params["lin"]["W"], params["lin"]["b"],
                           h1["W"], h1["b"])
    return (out, ll0, ll1, el0, el1, c0, c1)
